# Initial kernel scaffold; baseline (speedup 1.0000x reference)
import jax
import jax.numpy as jnp
from jax.experimental import pallas as pl


def kernel(x, base_weight, lora_A, lora_B):
    raise NotImplementedError("write your pallas kernel here")


# speedup vs baseline: 3.0671x; 3.0671x over previous
import functools

import jax
import jax.numpy as jnp
from jax import lax
from jax.experimental import pallas as pl
from jax.experimental.pallas import tpu as pltpu
from jax.experimental.pallas import tpu_sc as plsc

V = 1000000
D = 64
R = 16
N = 1024 * 200

NC = 2
NS = 16
NW = NC * NS
B_PER_W = N // NW
CHUNK = 640
NCHUNKS = B_PER_W // CHUNK

def _worker_id():
    return lax.axis_index("s") * NC + lax.axis_index("c")


@functools.cache
def _sc_kernels():
    mesh = plsc.VectorSubcoreMesh(core_axis_name="c", subcore_axis_name="s")

    @functools.partial(
        pl.kernel,
        out_type=jax.ShapeDtypeStruct((N, D), jnp.float32),
        mesh=mesh,
        compiler_params=pltpu.CompilerParams(use_tc_tiling_on_sc=False),
        scratch_types=[
            pltpu.VMEM((B_PER_W,), jnp.int32),
            pltpu.VMEM((CHUNK, D), jnp.float32),
            pltpu.VMEM((CHUNK, D), jnp.float32),
            pltpu.SemaphoreType.DMA,
            pltpu.SemaphoreType.DMA,
        ],
    )
    def base_gather(x_hbm, table_hbm, out_hbm, idx_v, buf0, buf1, sem0, sem1):
        base = _worker_id() * B_PER_W
        pltpu.sync_copy(x_hbm.at[pl.ds(base, B_PER_W)], idx_v)
        bufs = (buf0, buf1)
        sems = (sem0, sem1)
        cps = [None, None]
        cps[0] = pltpu.async_copy(table_hbm.at[idx_v.at[pl.ds(0, CHUNK)]], buf0, sem0)
        for k in range(NCHUNKS):
            if k + 1 < NCHUNKS:
                j = (k + 1) % 2
                cps[j] = pltpu.async_copy(
                    table_hbm.at[idx_v.at[pl.ds((k + 1) * CHUNK, CHUNK)]],
                    bufs[j],
                    sems[j],
                )
            cps[k % 2].wait()
            pltpu.sync_copy(bufs[k % 2], out_hbm.at[pl.ds(base + k * CHUNK, CHUNK)])

    @functools.partial(
        pl.kernel,
        out_type=jax.ShapeDtypeStruct((N, R), jnp.float32),
        mesh=mesh,
        compiler_params=pltpu.CompilerParams(use_tc_tiling_on_sc=False),
        scratch_types=[
            pltpu.VMEM((B_PER_W,), jnp.int32),
            pltpu.VMEM((B_PER_W, R), jnp.float32),
            pltpu.SemaphoreType.DMA,
        ],
    )
    def lora_gather(x_hbm, at_hbm, out_hbm, idx_v, rows_v, sem):
        base = _worker_id() * B_PER_W
        pltpu.sync_copy(x_hbm.at[pl.ds(base, B_PER_W)], idx_v)
        pltpu.async_copy(at_hbm.at[idx_v], rows_v, sem).wait()
        pltpu.sync_copy(rows_v, out_hbm.at[pl.ds(base, B_PER_W)])

    return base_gather, lora_gather


_VB = 2048


def _transpose_body(a_ref, out_ref):
    out_ref[...] = a_ref[...].T


_transpose = pl.pallas_call(
    _transpose_body,
    grid=(pl.cdiv(V, _VB),),
    in_specs=[pl.BlockSpec((R, _VB), lambda i: (0, i))],
    out_specs=pl.BlockSpec((_VB, R), lambda i: (i, 0)),
    out_shape=jax.ShapeDtypeStruct((V, R), jnp.float32),
)

_BN = 2048


def _fuse_body(ar_ref, rows_ref, b_ref, out_ref):
    out_ref[...] = rows_ref[...] + jnp.dot(
        ar_ref[...], b_ref[...].T, preferred_element_type=jnp.float32
    )


_fuse = pl.pallas_call(
    _fuse_body,
    grid=(N // _BN,),
    in_specs=[
        pl.BlockSpec((_BN, R), lambda i: (i, 0)),
        pl.BlockSpec((_BN, D), lambda i: (i, 0)),
        pl.BlockSpec((D, R), lambda i: (0, 0)),
    ],
    out_specs=pl.BlockSpec((_BN, D), lambda i: (i, 0)),
    out_shape=jax.ShapeDtypeStruct((N, D), jnp.float32),
)


def kernel(x, base_weight, lora_A, lora_B):
    Bsz, Ssz = x.shape
    x_flat = x.reshape(-1)
    base_gather, lora_gather = _sc_kernels()
    at = _transpose(lora_A)
    rows = base_gather(x_flat, base_weight)
    ar = lora_gather(x_flat, at)
    out = _fuse(ar, rows, lora_B)
    return out.reshape(Bsz, Ssz, D)
